# Initial kernel scaffold; baseline (speedup 1.0000x reference)
import jax
import jax.numpy as jnp
from jax.experimental import pallas as pl


def kernel(unit_features, unit_position_ids, table):
    raise NotImplementedError("write your pallas kernel here")


# speedup vs baseline: 2.2604x; 2.2604x over previous
import functools

import jax
import jax.numpy as jnp
from jax import lax
from jax.experimental import pallas as pl
from jax.experimental.pallas import tpu as pltpu
from jax.experimental.pallas import tpu_sc as plsc

N = 100000
D = 128
NC = 2
NS = 16
NW = NC * NS

CHUNK = 256
SUB = 128
NUM_FULL = N // CHUNK
TAIL = N - NUM_FULL * CHUNK
TAIL_BASE = NUM_FULL * CHUNK


def _body(feats_hbm, idx_hbm, table_hbm, out_hbm, idx_v, buf_v, sem):
    c = lax.axis_index("c")
    s = lax.axis_index("s")
    wid = s * NC + c

    def do_chunk(base):
        base = pl.multiple_of(base, CHUNK)
        pltpu.sync_copy(feats_hbm.at[pl.ds(base, CHUNK)], buf_v)
        pltpu.sync_copy(idx_hbm.at[pl.ds(base, CHUNK)], idx_v)
        copies = []
        for j in range(CHUNK // SUB):
            copies.append(pltpu.async_copy(
                table_hbm.at[idx_v.at[pl.ds(j * SUB, SUB)]],
                buf_v.at[pl.ds(j * SUB, SUB)],
                sem, add=True))
        for cp in copies:
            cp.wait()
        pltpu.sync_copy(buf_v, out_hbm.at[pl.ds(base, CHUNK)])

    n_my = (NUM_FULL - wid + NW - 1) // NW

    def loop_body(i, carry):
        do_chunk((wid + i * NW) * CHUNK)
        return carry

    lax.fori_loop(0, n_my, loop_body, 0)

    @pl.when(wid == NW - 1)
    def _tail():
        pltpu.sync_copy(feats_hbm.at[pl.ds(TAIL_BASE, TAIL)],
                        buf_v.at[pl.ds(0, TAIL)])
        pltpu.sync_copy(idx_hbm.at[pl.ds(TAIL_BASE, TAIL)],
                        idx_v.at[pl.ds(0, TAIL)])
        cp1 = pltpu.async_copy(table_hbm.at[idx_v.at[pl.ds(0, SUB)]],
                               buf_v.at[pl.ds(0, SUB)], sem, add=True)
        cp2 = pltpu.async_copy(table_hbm.at[idx_v.at[pl.ds(SUB, TAIL - SUB)]],
                               buf_v.at[pl.ds(SUB, TAIL - SUB)], sem, add=True)
        cp1.wait()
        cp2.wait()
        pltpu.sync_copy(buf_v.at[pl.ds(0, TAIL)],
                        out_hbm.at[pl.ds(TAIL_BASE, TAIL)])


@functools.partial(jax.jit, donate_argnums=())
def _run(feats, idx, table):
    mesh = plsc.VectorSubcoreMesh(core_axis_name="c", subcore_axis_name="s",
                                  num_cores=NC, num_subcores=NS)
    return pl.kernel(
        _body,
        out_type=jax.ShapeDtypeStruct((N, D), jnp.float32),
        mesh=mesh,
        scratch_types=[
            pltpu.VMEM((CHUNK,), jnp.int32),
            pltpu.VMEM((CHUNK, D), jnp.float32),
            pltpu.SemaphoreType.DMA,
        ],
    )(feats, idx, table)


def kernel(unit_features, unit_position_ids, table):
    idx = unit_position_ids.astype(jnp.int32)
    return _run(unit_features, idx, table)


# speedup vs baseline: 2.4486x; 1.0832x over previous
import functools

import jax
import jax.numpy as jnp
from jax import lax
from jax.experimental import pallas as pl
from jax.experimental.pallas import tpu as pltpu
from jax.experimental.pallas import tpu_sc as plsc

N = 100000
D = 128
NC = 2
NS = 16
NW = NC * NS

CHUNK = 256
SUB = 128
NUM_FULL = N // CHUNK
TAIL = N - NUM_FULL * CHUNK
TAIL_BASE = NUM_FULL * CHUNK
BASE_CHUNKS = NUM_FULL // NW
EXTRA = NUM_FULL - BASE_CHUNKS * NW
IDX_CAP = (BASE_CHUNKS + 1) * CHUNK


def _body(feats_hbm, idx_hbm, table_hbm, out_hbm, idx_v, buf_v, load_sems,
          store_sems, gather_sem):
    c = lax.axis_index("c")
    s = lax.axis_index("s")
    wid = s * NC + c

    start_chunk = BASE_CHUNKS * wid + jnp.minimum(wid, EXTRA)
    row0 = pl.multiple_of(start_chunk * CHUNK, CHUNK)
    n_my = BASE_CHUNKS + jnp.where(wid < EXTRA, 1, 0)

    pltpu.sync_copy(idx_hbm.at[pl.ds(row0, BASE_CHUNKS * CHUNK)],
                    idx_v.at[pl.ds(0, BASE_CHUNKS * CHUNK)])

    @pl.when(wid < EXTRA)
    def _extra_idx():
        pltpu.sync_copy(
            idx_hbm.at[pl.ds(row0 + BASE_CHUNKS * CHUNK, CHUNK)],
            idx_v.at[pl.ds(BASE_CHUNKS * CHUNK, CHUNK)])

    @pl.when(wid == NW - 1)
    def _tail_idx():
        pltpu.sync_copy(idx_hbm.at[pl.ds(TAIL_BASE, TAIL)],
                        idx_v.at[pl.ds(BASE_CHUNKS * CHUNK, TAIL)])

    def chunk_base(k):
        return pl.multiple_of((start_chunk + k) * CHUNK, CHUNK)

    def load_start(k, b):
        pltpu.async_copy(feats_hbm.at[pl.ds(chunk_base(k), CHUNK)],
                         buf_v.at[b], load_sems.at[b])

    def load_wait(k, b):
        pltpu.make_async_copy(feats_hbm.at[pl.ds(chunk_base(k), CHUNK)],
                              buf_v.at[b], load_sems.at[b]).wait()

    def store_start(k, b):
        pltpu.async_copy(buf_v.at[b], out_hbm.at[pl.ds(chunk_base(k), CHUNK)],
                         store_sems.at[b])

    def store_wait(k, b):
        pltpu.make_async_copy(buf_v.at[b],
                              out_hbm.at[pl.ds(chunk_base(k), CHUNK)],
                              store_sems.at[b]).wait()

    load_start(0, 0)

    def loop_body(i, carry):
        b = lax.rem(i, 2)
        nb = 1 - b

        @pl.when(i >= 1)
        def _drain_prev_store():
            store_wait(i - 1, nb)

        @pl.when(i + 1 < n_my)
        def _next_load():
            load_start(i + 1, nb)

        load_wait(i, b)
        cps = []
        for j in range(CHUNK // SUB):
            cps.append(pltpu.async_copy(
                table_hbm.at[idx_v.at[pl.ds(i * CHUNK + j * SUB, SUB)]],
                buf_v.at[b].at[pl.ds(j * SUB, SUB)],
                gather_sem, add=True))
        for cp in cps:
            cp.wait()
        store_start(i, b)
        return carry

    lax.fori_loop(0, n_my, loop_body, 0)
    store_wait(n_my - 1, lax.rem(n_my - 1, 2))

    @pl.when(wid == NW - 1)
    def _tail():
        tbuf = buf_v.at[0]
        pltpu.sync_copy(feats_hbm.at[pl.ds(TAIL_BASE, TAIL)],
                        tbuf.at[pl.ds(0, TAIL)])
        cp1 = pltpu.async_copy(
            table_hbm.at[idx_v.at[pl.ds(BASE_CHUNKS * CHUNK, SUB)]],
            tbuf.at[pl.ds(0, SUB)], gather_sem, add=True)
        cp2 = pltpu.async_copy(
            table_hbm.at[idx_v.at[pl.ds(BASE_CHUNKS * CHUNK + SUB, TAIL - SUB)]],
            tbuf.at[pl.ds(SUB, TAIL - SUB)], gather_sem, add=True)
        cp1.wait()
        cp2.wait()
        pltpu.sync_copy(tbuf.at[pl.ds(0, TAIL)],
                        out_hbm.at[pl.ds(TAIL_BASE, TAIL)])


@jax.jit
def _run(feats, idx, table):
    mesh = plsc.VectorSubcoreMesh(core_axis_name="c", subcore_axis_name="s",
                                  num_cores=NC, num_subcores=NS)
    return pl.kernel(
        _body,
        out_type=jax.ShapeDtypeStruct((N, D), jnp.float32),
        mesh=mesh,
        scratch_types=[
            pltpu.VMEM((IDX_CAP,), jnp.int32),
            pltpu.VMEM((2, CHUNK, D), jnp.float32),
            pltpu.SemaphoreType.DMA((2,)),
            pltpu.SemaphoreType.DMA((2,)),
            pltpu.SemaphoreType.DMA,
        ],
    )(feats, idx, table)


def kernel(unit_features, unit_position_ids, table):
    idx = unit_position_ids.astype(jnp.int32)
    return _run(unit_features, idx, table)


# speedup vs baseline: 2.5182x; 1.0284x over previous
import functools

import jax
import jax.numpy as jnp
from jax import lax
from jax.experimental import pallas as pl
from jax.experimental.pallas import tpu as pltpu
from jax.experimental.pallas import tpu_sc as plsc

N = 100000
D = 128
NC = 2
NS = 16
NW = NC * NS
NBUF = 3

CHUNK = 256
SUB = 128
NUM_FULL = N // CHUNK
TAIL = N - NUM_FULL * CHUNK
TAIL_BASE = NUM_FULL * CHUNK
BASE_CHUNKS = NUM_FULL // NW
EXTRA = NUM_FULL - BASE_CHUNKS * NW
IDX_CAP = (BASE_CHUNKS + 1) * CHUNK


def _body(feats_hbm, idx_hbm, table_hbm, out_hbm, idx_v, buf_v, load_sems,
          store_sems, gather_sem):
    c = lax.axis_index("c")
    s = lax.axis_index("s")
    wid = s * NC + c

    start_chunk = BASE_CHUNKS * wid + jnp.minimum(wid, EXTRA)
    row0 = pl.multiple_of(start_chunk * CHUNK, CHUNK)
    n_my = BASE_CHUNKS + jnp.where(wid < EXTRA, 1, 0)

    pltpu.sync_copy(idx_hbm.at[pl.ds(row0, BASE_CHUNKS * CHUNK)],
                    idx_v.at[pl.ds(0, BASE_CHUNKS * CHUNK)])

    @pl.when(wid < EXTRA)
    def _extra_idx():
        pltpu.sync_copy(
            idx_hbm.at[pl.ds(row0 + BASE_CHUNKS * CHUNK, CHUNK)],
            idx_v.at[pl.ds(BASE_CHUNKS * CHUNK, CHUNK)])

    @pl.when(wid == NW - 1)
    def _tail_idx():
        pltpu.sync_copy(idx_hbm.at[pl.ds(TAIL_BASE, TAIL)],
                        idx_v.at[pl.ds(BASE_CHUNKS * CHUNK, TAIL)])

    def chunk_base(k):
        return pl.multiple_of((start_chunk + k) * CHUNK, CHUNK)

    def load_start(k, b):
        pltpu.async_copy(feats_hbm.at[pl.ds(chunk_base(k), CHUNK)],
                         buf_v.at[b], load_sems.at[b])

    def load_wait(k, b):
        pltpu.make_async_copy(feats_hbm.at[pl.ds(chunk_base(k), CHUNK)],
                              buf_v.at[b], load_sems.at[b]).wait()

    def store_start(k, b):
        pltpu.async_copy(buf_v.at[b], out_hbm.at[pl.ds(chunk_base(k), CHUNK)],
                         store_sems.at[b])

    def store_wait(k, b):
        pltpu.make_async_copy(buf_v.at[b],
                              out_hbm.at[pl.ds(chunk_base(k), CHUNK)],
                              store_sems.at[b]).wait()

    load_start(0, 0)
    load_start(1, 1)

    def loop_body(i, carry):
        b = lax.rem(i, NBUF)

        load_wait(i, b)
        cps = []
        for j in range(CHUNK // SUB):
            cps.append(pltpu.async_copy(
                table_hbm.at[idx_v.at[pl.ds(i * CHUNK + j * SUB, SUB)]],
                buf_v.at[b].at[pl.ds(j * SUB, SUB)],
                gather_sem, add=True))

        @pl.when(i >= 1)
        def _drain_prev_store():
            store_wait(i - 1, lax.rem(i - 1, NBUF))

        @pl.when(i + 2 < n_my)
        def _next_load():
            load_start(i + 2, lax.rem(i + 2, NBUF))

        for cp in cps:
            cp.wait()
        store_start(i, b)
        return carry

    lax.fori_loop(0, n_my, loop_body, 0)
    store_wait(n_my - 1, lax.rem(n_my - 1, NBUF))

    @pl.when(wid == NW - 1)
    def _tail():
        tbuf = buf_v.at[0]
        pltpu.sync_copy(feats_hbm.at[pl.ds(TAIL_BASE, TAIL)],
                        tbuf.at[pl.ds(0, TAIL)])
        cp1 = pltpu.async_copy(
            table_hbm.at[idx_v.at[pl.ds(BASE_CHUNKS * CHUNK, SUB)]],
            tbuf.at[pl.ds(0, SUB)], gather_sem, add=True)
        cp2 = pltpu.async_copy(
            table_hbm.at[idx_v.at[pl.ds(BASE_CHUNKS * CHUNK + SUB, TAIL - SUB)]],
            tbuf.at[pl.ds(SUB, TAIL - SUB)], gather_sem, add=True)
        cp1.wait()
        cp2.wait()
        pltpu.sync_copy(tbuf.at[pl.ds(0, TAIL)],
                        out_hbm.at[pl.ds(TAIL_BASE, TAIL)])


@jax.jit
def _run(feats, idx, table):
    mesh = plsc.VectorSubcoreMesh(core_axis_name="c", subcore_axis_name="s",
                                  num_cores=NC, num_subcores=NS)
    return pl.kernel(
        _body,
        out_type=jax.ShapeDtypeStruct((N, D), jnp.float32),
        mesh=mesh,
        scratch_types=[
            pltpu.VMEM((IDX_CAP,), jnp.int32),
            pltpu.VMEM((NBUF, CHUNK, D), jnp.float32),
            pltpu.SemaphoreType.DMA((NBUF,)),
            pltpu.SemaphoreType.DMA((NBUF,)),
            pltpu.SemaphoreType.DMA,
        ],
    )(feats, idx, table)


def kernel(unit_features, unit_position_ids, table):
    idx = unit_position_ids.astype(jnp.int32)
    return _run(unit_features, idx, table)


# speedup vs baseline: 4.0307x; 1.6006x over previous
import functools

import jax
import jax.numpy as jnp
from jax import lax
from jax.experimental import pallas as pl
from jax.experimental.pallas import tpu as pltpu
from jax.experimental.pallas import tpu_sc as plsc

N = 100000
D = 128
NC = 2
NS = 16
NW = NC * NS
NBUF = 3

CHUNK = 256
SUB = 128
NUM_FULL = N // CHUNK
TAIL = N - NUM_FULL * CHUNK
TAIL_BASE = NUM_FULL * CHUNK
BASE_CHUNKS = NUM_FULL // NW
EXTRA = NUM_FULL - BASE_CHUNKS * NW
IDX_CAP = (BASE_CHUNKS + 1) * CHUNK


def _body(feats_hbm, idx_hbm, table_hbm, out_hbm, idx_v, buf_v, load_sems,
          store_sems, gather_sem, table_sp):
    c = lax.axis_index("c")
    s = lax.axis_index("s")
    wid = s * NC + c

    @pl.when(s == 0)
    def _stage_table():
        pltpu.sync_copy(table_hbm, table_sp)

    start_chunk = BASE_CHUNKS * wid + jnp.minimum(wid, EXTRA)
    row0 = pl.multiple_of(start_chunk * CHUNK, CHUNK)
    n_my = BASE_CHUNKS + jnp.where(wid < EXTRA, 1, 0)

    pltpu.sync_copy(idx_hbm.at[pl.ds(row0, BASE_CHUNKS * CHUNK)],
                    idx_v.at[pl.ds(0, BASE_CHUNKS * CHUNK)])

    @pl.when(wid < EXTRA)
    def _extra_idx():
        pltpu.sync_copy(
            idx_hbm.at[pl.ds(row0 + BASE_CHUNKS * CHUNK, CHUNK)],
            idx_v.at[pl.ds(BASE_CHUNKS * CHUNK, CHUNK)])

    @pl.when(wid == NW - 1)
    def _tail_idx():
        pltpu.sync_copy(idx_hbm.at[pl.ds(TAIL_BASE, TAIL)],
                        idx_v.at[pl.ds(BASE_CHUNKS * CHUNK, TAIL)])

    plsc.subcore_barrier()

    def chunk_base(k):
        return pl.multiple_of((start_chunk + k) * CHUNK, CHUNK)

    def load_start(k, b):
        pltpu.async_copy(feats_hbm.at[pl.ds(chunk_base(k), CHUNK)],
                         buf_v.at[b], load_sems.at[b])

    def load_wait(k, b):
        pltpu.make_async_copy(feats_hbm.at[pl.ds(chunk_base(k), CHUNK)],
                              buf_v.at[b], load_sems.at[b]).wait()

    def store_start(k, b):
        pltpu.async_copy(buf_v.at[b], out_hbm.at[pl.ds(chunk_base(k), CHUNK)],
                         store_sems.at[b])

    def store_wait(k, b):
        pltpu.make_async_copy(buf_v.at[b],
                              out_hbm.at[pl.ds(chunk_base(k), CHUNK)],
                              store_sems.at[b]).wait()

    load_start(0, 0)
    load_start(1, 1)

    def loop_body(i, carry):
        b = lax.rem(i, NBUF)

        load_wait(i, b)
        cps = []
        for j in range(CHUNK // SUB):
            cps.append(pltpu.async_copy(
                table_sp.at[idx_v.at[pl.ds(i * CHUNK + j * SUB, SUB)]],
                buf_v.at[b].at[pl.ds(j * SUB, SUB)],
                gather_sem, add=True))

        @pl.when(i >= 1)
        def _drain_prev_store():
            store_wait(i - 1, lax.rem(i - 1, NBUF))

        @pl.when(i + 2 < n_my)
        def _next_load():
            load_start(i + 2, lax.rem(i + 2, NBUF))

        for cp in cps:
            cp.wait()
        store_start(i, b)
        return carry

    lax.fori_loop(0, n_my, loop_body, 0)
    store_wait(n_my - 1, lax.rem(n_my - 1, NBUF))

    @pl.when(wid == NW - 1)
    def _tail():
        tbuf = buf_v.at[0]
        pltpu.sync_copy(feats_hbm.at[pl.ds(TAIL_BASE, TAIL)],
                        tbuf.at[pl.ds(0, TAIL)])
        cp1 = pltpu.async_copy(
            table_sp.at[idx_v.at[pl.ds(BASE_CHUNKS * CHUNK, SUB)]],
            tbuf.at[pl.ds(0, SUB)], gather_sem, add=True)
        cp2 = pltpu.async_copy(
            table_sp.at[idx_v.at[pl.ds(BASE_CHUNKS * CHUNK + SUB, TAIL - SUB)]],
            tbuf.at[pl.ds(SUB, TAIL - SUB)], gather_sem, add=True)
        cp1.wait()
        cp2.wait()
        pltpu.sync_copy(tbuf.at[pl.ds(0, TAIL)],
                        out_hbm.at[pl.ds(TAIL_BASE, TAIL)])


@jax.jit
def _run(feats, idx, table):
    mesh = plsc.VectorSubcoreMesh(core_axis_name="c", subcore_axis_name="s",
                                  num_cores=NC, num_subcores=NS)
    return pl.kernel(
        _body,
        out_type=jax.ShapeDtypeStruct((N, D), jnp.float32),
        mesh=mesh,
        scratch_types=[
            pltpu.VMEM((IDX_CAP,), jnp.int32),
            pltpu.VMEM((NBUF, CHUNK, D), jnp.float32),
            pltpu.SemaphoreType.DMA((NBUF,)),
            pltpu.SemaphoreType.DMA((NBUF,)),
            pltpu.SemaphoreType.DMA,
            pltpu.VMEM_SHARED((1024, D), jnp.float32),
        ],
    )(feats, idx, table)


def kernel(unit_features, unit_position_ids, table):
    idx = unit_position_ids.astype(jnp.int32)
    return _run(unit_features, idx, table)


# speedup vs baseline: 4.1002x; 1.0172x over previous
import functools

import jax
import jax.numpy as jnp
from jax import lax
from jax.experimental import pallas as pl
from jax.experimental.pallas import tpu as pltpu
from jax.experimental.pallas import tpu_sc as plsc

N = 100000
D = 128
NC = 2
NS = 16
NW = NC * NS
NBUF = 3

CHUNK = 256
SUB = 128
NUM_FULL = N // CHUNK
TAIL = N - NUM_FULL * CHUNK
TAIL_BASE = NUM_FULL * CHUNK
BASE_CHUNKS = NUM_FULL // NW
EXTRA = NUM_FULL - BASE_CHUNKS * NW
IDX_CAP = (BASE_CHUNKS + 1) * CHUNK


def _body(feats_hbm, idx_hbm, table_hbm, out_hbm, idx_v, buf_v, load_sems,
          store_sems, gather_sem, table_sp):
    c = lax.axis_index("c")
    s = lax.axis_index("s")
    wid = s * NC + c

    start_chunk = BASE_CHUNKS * wid + jnp.minimum(wid, EXTRA)
    row0 = pl.multiple_of(start_chunk * CHUNK, CHUNK)
    n_my = BASE_CHUNKS + jnp.where(wid < EXTRA, 1, 0)

    @pl.when(wid < EXTRA)
    def _extra_idx():
        pltpu.sync_copy(
            idx_hbm.at[pl.ds(row0 + BASE_CHUNKS * CHUNK, CHUNK)],
            idx_v.at[pl.ds(BASE_CHUNKS * CHUNK, CHUNK)])

    @pl.when(wid == NW - 1)
    def _tail_idx():
        pltpu.sync_copy(idx_hbm.at[pl.ds(TAIL_BASE, TAIL)],
                        idx_v.at[pl.ds(BASE_CHUNKS * CHUNK, TAIL)])

    def chunk_base(k):
        return pl.multiple_of((start_chunk + k) * CHUNK, CHUNK)

    def load_start(k, b):
        pltpu.async_copy(feats_hbm.at[pl.ds(chunk_base(k), CHUNK)],
                         buf_v.at[b], load_sems.at[b])

    def load_wait(k, b):
        pltpu.make_async_copy(feats_hbm.at[pl.ds(chunk_base(k), CHUNK)],
                              buf_v.at[b], load_sems.at[b]).wait()

    def store_start(k, b):
        pltpu.async_copy(buf_v.at[b], out_hbm.at[pl.ds(chunk_base(k), CHUNK)],
                         store_sems.at[b])

    def store_wait(k, b):
        pltpu.make_async_copy(buf_v.at[b],
                              out_hbm.at[pl.ds(chunk_base(k), CHUNK)],
                              store_sems.at[b]).wait()

    load_start(0, 0)
    load_start(1, 1)

    pltpu.sync_copy(idx_hbm.at[pl.ds(row0, BASE_CHUNKS * CHUNK)],
                    idx_v.at[pl.ds(0, BASE_CHUNKS * CHUNK)])

    @pl.when(s == 0)
    def _stage_table():
        pltpu.sync_copy(table_hbm, table_sp)

    plsc.subcore_barrier()

    def loop_body(i, carry):
        b = lax.rem(i, NBUF)

        load_wait(i, b)
        cps = []
        for j in range(CHUNK // SUB):
            cps.append(pltpu.async_copy(
                table_sp.at[idx_v.at[pl.ds(i * CHUNK + j * SUB, SUB)]],
                buf_v.at[b].at[pl.ds(j * SUB, SUB)],
                gather_sem, add=True))

        @pl.when(i >= 1)
        def _drain_prev_store():
            store_wait(i - 1, lax.rem(i - 1, NBUF))

        @pl.when(i + 2 < n_my)
        def _next_load():
            load_start(i + 2, lax.rem(i + 2, NBUF))

        for cp in cps:
            cp.wait()
        store_start(i, b)
        return carry

    lax.fori_loop(0, n_my, loop_body, 0)
    store_wait(n_my - 1, lax.rem(n_my - 1, NBUF))

    @pl.when(wid == NW - 1)
    def _tail():
        tbuf = buf_v.at[0]
        pltpu.sync_copy(feats_hbm.at[pl.ds(TAIL_BASE, TAIL)],
                        tbuf.at[pl.ds(0, TAIL)])
        cp1 = pltpu.async_copy(
            table_sp.at[idx_v.at[pl.ds(BASE_CHUNKS * CHUNK, SUB)]],
            tbuf.at[pl.ds(0, SUB)], gather_sem, add=True)
        cp2 = pltpu.async_copy(
            table_sp.at[idx_v.at[pl.ds(BASE_CHUNKS * CHUNK + SUB, TAIL - SUB)]],
            tbuf.at[pl.ds(SUB, TAIL - SUB)], gather_sem, add=True)
        cp1.wait()
        cp2.wait()
        pltpu.sync_copy(tbuf.at[pl.ds(0, TAIL)],
                        out_hbm.at[pl.ds(TAIL_BASE, TAIL)])


@jax.jit
def _run(feats, idx, table):
    mesh = plsc.VectorSubcoreMesh(core_axis_name="c", subcore_axis_name="s",
                                  num_cores=NC, num_subcores=NS)
    return pl.kernel(
        _body,
        out_type=jax.ShapeDtypeStruct((N, D), jnp.float32),
        mesh=mesh,
        scratch_types=[
            pltpu.VMEM((IDX_CAP,), jnp.int32),
            pltpu.VMEM((NBUF, CHUNK, D), jnp.float32),
            pltpu.SemaphoreType.DMA((NBUF,)),
            pltpu.SemaphoreType.DMA((NBUF,)),
            pltpu.SemaphoreType.DMA,
            pltpu.VMEM_SHARED((1024, D), jnp.float32),
        ],
    )(feats, idx, table)


def kernel(unit_features, unit_position_ids, table):
    idx = unit_position_ids.astype(jnp.int32)
    return _run(unit_features, idx, table)


# speedup vs baseline: 4.2475x; 1.0359x over previous
import jax
import jax.numpy as jnp
from jax import lax
from jax.experimental import pallas as pl
from jax.experimental.pallas import tpu as pltpu
from jax.experimental.pallas import tpu_sc as plsc

N = 100000
D = 128
P = 1024
NC = 2
NS = 16
NW = NC * NS
NBUF = 3

CHUNK = 248
ROWS_MAIN = 3128
ROWS_LAST = N - (NW - 1) * ROWS_MAIN
NCHUNKS = 12
TAIL_OFF = NCHUNKS * CHUNK
TAIL_MAIN = ROWS_MAIN - TAIL_OFF
TAIL_LAST = ROWS_LAST - TAIL_OFF
IDX_CAP = ROWS_MAIN + 8

def _subsplits(total):
    offs, o = [], 0
    while o < total:
        sz = min(128, total - o)
        offs.append((o, sz))
        o += sz
    return offs


def _body(feats_hbm, idx_hbm, table_hbm, out_hbm, idx_v, buf_v, tail_v,
          load_sems, store_sems, gather_sem, tail_sem, tail_gsem, table_sp):
    c = lax.axis_index("c")
    s = lax.axis_index("s")
    wid = s * NC + c
    is_last = wid == NW - 1

    row0 = pl.multiple_of(wid * ROWS_MAIN, 8)

    def chunk_base(k):
        return pl.multiple_of(row0 + k * CHUNK, 8)

    def load_start(k, b):
        pltpu.async_copy(feats_hbm.at[pl.ds(chunk_base(k), CHUNK)],
                         buf_v.at[b], load_sems.at[b])

    def load_wait(k, b):
        pltpu.make_async_copy(feats_hbm.at[pl.ds(chunk_base(k), CHUNK)],
                              buf_v.at[b], load_sems.at[b]).wait()

    def store_start(k, b):
        pltpu.async_copy(buf_v.at[b], out_hbm.at[pl.ds(chunk_base(k), CHUNK)],
                         store_sems.at[b])

    def store_wait(k, b):
        pltpu.make_async_copy(buf_v.at[b],
                              out_hbm.at[pl.ds(chunk_base(k), CHUNK)],
                              store_sems.at[b]).wait()

    def tail_feat_copy(tail_rows):
        return pltpu.make_async_copy(
            feats_hbm.at[pl.ds(row0 + TAIL_OFF, tail_rows)],
            tail_v.at[pl.ds(0, tail_rows)], tail_sem)

    def tail_gathers(tail_rows):
        return [pltpu.make_async_copy(
                    table_sp.at[idx_v.at[pl.ds(TAIL_OFF + o, sz)]],
                    tail_v.at[pl.ds(o, sz)], tail_gsem)
                for o, sz in _subsplits(tail_rows)]

    def tail_out_copy(tail_rows):
        return pltpu.make_async_copy(
            tail_v.at[pl.ds(0, tail_rows)],
            out_hbm.at[pl.ds(row0 + TAIL_OFF, tail_rows)], tail_sem)

    load_start(0, 0)
    load_start(1, 1)

    @pl.when(~is_last)
    def _tail_load_main():
        tail_feat_copy(TAIL_MAIN).start()

    @pl.when(is_last)
    def _tail_load_last():
        tail_feat_copy(TAIL_LAST).start()

    @pl.when(~is_last)
    def _idx_main():
        pltpu.sync_copy(idx_hbm.at[pl.ds(row0, ROWS_MAIN)],
                        idx_v.at[pl.ds(0, ROWS_MAIN)])

    @pl.when(is_last)
    def _idx_last():
        pltpu.sync_copy(idx_hbm.at[pl.ds(row0, ROWS_LAST)],
                        idx_v.at[pl.ds(0, ROWS_LAST)])

    @pl.when(s == 0)
    def _stage_table():
        pltpu.sync_copy(table_hbm, table_sp)

    plsc.subcore_barrier()

    @pl.when(~is_last)
    def _tail_gather_main():
        tail_feat_copy(TAIL_MAIN).wait()
        for g in tail_gathers(TAIL_MAIN):
            pltpu.async_copy(g.src_ref, g.dst_ref, tail_gsem, add=True)

    @pl.when(is_last)
    def _tail_gather_last():
        tail_feat_copy(TAIL_LAST).wait()
        for g in tail_gathers(TAIL_LAST):
            pltpu.async_copy(g.src_ref, g.dst_ref, tail_gsem, add=True)

    def loop_body(i, carry):
        b = lax.rem(i, NBUF)

        load_wait(i, b)
        cps = []
        for o, sz in _subsplits(CHUNK):
            cps.append(pltpu.async_copy(
                table_sp.at[idx_v.at[pl.ds(i * CHUNK + o, sz)]],
                buf_v.at[b].at[pl.ds(o, sz)],
                gather_sem, add=True))

        @pl.when(i >= 1)
        def _drain_prev_store():
            store_wait(i - 1, lax.rem(i - 1, NBUF))

        @pl.when(i + 2 < NCHUNKS)
        def _next_load():
            load_start(i + 2, lax.rem(i + 2, NBUF))

        for cp in cps:
            cp.wait()
        store_start(i, b)
        return carry

    lax.fori_loop(0, NCHUNKS, loop_body, 0)

    @pl.when(~is_last)
    def _tail_finish_main():
        for g in tail_gathers(TAIL_MAIN):
            g.wait()
        tail_out_copy(TAIL_MAIN).start()

    @pl.when(is_last)
    def _tail_finish_last():
        for g in tail_gathers(TAIL_LAST):
            g.wait()
        tail_out_copy(TAIL_LAST).start()

    store_wait(NCHUNKS - 1, lax.rem(NCHUNKS - 1, NBUF))

    @pl.when(~is_last)
    def _tail_drain_main():
        tail_out_copy(TAIL_MAIN).wait()

    @pl.when(is_last)
    def _tail_drain_last():
        tail_out_copy(TAIL_LAST).wait()


@jax.jit
def _run(feats, idx, table):
    mesh = plsc.VectorSubcoreMesh(core_axis_name="c", subcore_axis_name="s",
                                  num_cores=NC, num_subcores=NS)
    return pl.kernel(
        _body,
        out_type=jax.ShapeDtypeStruct((N, D), jnp.float32),
        mesh=mesh,
        scratch_types=[
            pltpu.VMEM((IDX_CAP,), jnp.int32),
            pltpu.VMEM((NBUF, CHUNK, D), jnp.float32),
            pltpu.VMEM((TAIL_MAIN, D), jnp.float32),
            pltpu.SemaphoreType.DMA((NBUF,)),
            pltpu.SemaphoreType.DMA((NBUF,)),
            pltpu.SemaphoreType.DMA,
            pltpu.SemaphoreType.DMA,
            pltpu.SemaphoreType.DMA,
            pltpu.VMEM_SHARED((P, D), jnp.float32),
        ],
    )(feats, idx, table)


def kernel(unit_features, unit_position_ids, table):
    idx = unit_position_ids.astype(jnp.int32)
    return _run(unit_features, idx, table)


# speedup vs baseline: 4.3367x; 1.0210x over previous
import jax
import jax.numpy as jnp
from jax import lax
from jax.experimental import pallas as pl
from jax.experimental.pallas import tpu as pltpu
from jax.experimental.pallas import tpu_sc as plsc

N = 100000
D = 128
P = 1024
NC = 2
NS = 16
NW = NC * NS
NBUF = 4

CHUNK = 184
ROWS_MAIN = 3128
ROWS_LAST = N - (NW - 1) * ROWS_MAIN
NCHUNKS_MAIN = ROWS_MAIN // CHUNK
NCHUNKS_LAST = ROWS_LAST // CHUNK
TAIL_OFF = NCHUNKS_LAST * CHUNK
TAIL_LAST = ROWS_LAST - TAIL_OFF
IDX_CAP = ROWS_MAIN + 8

def _subsplits(total):
    offs, o = [], 0
    while o < total:
        sz = min(128, total - o)
        offs.append((o, sz))
        o += sz
    return offs


def _body(feats_hbm, idx_hbm, table_hbm, out_hbm, idx_v, buf_v, tail_v,
          load_sems, store_sems, gather_sem, tail_sem, tail_gsem, table_sp):
    c = lax.axis_index("c")
    s = lax.axis_index("s")
    wid = s * NC + c
    is_last = wid == NW - 1

    row0 = pl.multiple_of(wid * ROWS_MAIN, 8)
    n_my = jnp.where(is_last, NCHUNKS_LAST, NCHUNKS_MAIN)

    def chunk_base(k):
        return pl.multiple_of(row0 + k * CHUNK, 8)

    def load_start(k, b):
        pltpu.async_copy(feats_hbm.at[pl.ds(chunk_base(k), CHUNK)],
                         buf_v.at[b], load_sems.at[b])

    def load_wait(k, b):
        pltpu.make_async_copy(feats_hbm.at[pl.ds(chunk_base(k), CHUNK)],
                              buf_v.at[b], load_sems.at[b]).wait()

    def store_start(k, b):
        pltpu.async_copy(buf_v.at[b], out_hbm.at[pl.ds(chunk_base(k), CHUNK)],
                         store_sems.at[b])

    def store_wait(k, b):
        pltpu.make_async_copy(buf_v.at[b],
                              out_hbm.at[pl.ds(chunk_base(k), CHUNK)],
                              store_sems.at[b]).wait()

    def tail_feat_copy():
        return pltpu.make_async_copy(
            feats_hbm.at[pl.ds(row0 + TAIL_OFF, TAIL_LAST)],
            tail_v.at[pl.ds(0, TAIL_LAST)], tail_sem)

    def tail_gathers():
        return [pltpu.make_async_copy(
                    table_sp.at[idx_v.at[pl.ds(TAIL_OFF + o, sz)]],
                    tail_v.at[pl.ds(o, sz)], tail_gsem)
                for o, sz in _subsplits(TAIL_LAST)]

    def tail_out_copy():
        return pltpu.make_async_copy(
            tail_v.at[pl.ds(0, TAIL_LAST)],
            out_hbm.at[pl.ds(row0 + TAIL_OFF, TAIL_LAST)], tail_sem)

    load_start(0, 0)
    load_start(1, 1)
    load_start(2, 2)

    @pl.when(is_last)
    def _tail_load_last():
        tail_feat_copy().start()

    @pl.when(~is_last)
    def _idx_main():
        pltpu.sync_copy(idx_hbm.at[pl.ds(row0, ROWS_MAIN)],
                        idx_v.at[pl.ds(0, ROWS_MAIN)])

    @pl.when(is_last)
    def _idx_last():
        pltpu.sync_copy(idx_hbm.at[pl.ds(row0, ROWS_LAST)],
                        idx_v.at[pl.ds(0, ROWS_LAST)])

    @pl.when(s == 0)
    def _stage_table():
        pltpu.sync_copy(table_hbm, table_sp)

    plsc.subcore_barrier()

    @pl.when(is_last)
    def _tail_gather_last():
        tail_feat_copy().wait()
        for g in tail_gathers():
            pltpu.async_copy(g.src_ref, g.dst_ref, tail_gsem, add=True)

    def loop_body(i, carry):
        b = lax.rem(i, NBUF)

        load_wait(i, b)
        cps = []
        for o, sz in _subsplits(CHUNK):
            cps.append(pltpu.async_copy(
                table_sp.at[idx_v.at[pl.ds(i * CHUNK + o, sz)]],
                buf_v.at[b].at[pl.ds(o, sz)],
                gather_sem, add=True))

        @pl.when(i >= 1)
        def _drain_prev_store():
            store_wait(i - 1, lax.rem(i - 1, NBUF))

        @pl.when(i + 3 < n_my)
        def _next_load():
            load_start(i + 3, lax.rem(i + 3, NBUF))

        for cp in cps:
            cp.wait()
        store_start(i, b)
        return carry

    lax.fori_loop(0, n_my, loop_body, 0)

    @pl.when(is_last)
    def _tail_finish_last():
        for g in tail_gathers():
            g.wait()
        tail_out_copy().start()

    store_wait(n_my - 1, lax.rem(n_my - 1, NBUF))

    @pl.when(is_last)
    def _tail_drain_last():
        tail_out_copy().wait()


@jax.jit
def _run(feats, idx, table):
    mesh = plsc.VectorSubcoreMesh(core_axis_name="c", subcore_axis_name="s",
                                  num_cores=NC, num_subcores=NS)
    return pl.kernel(
        _body,
        out_type=jax.ShapeDtypeStruct((N, D), jnp.float32),
        mesh=mesh,
        scratch_types=[
            pltpu.VMEM((IDX_CAP,), jnp.int32),
            pltpu.VMEM((NBUF, CHUNK, D), jnp.float32),
            pltpu.VMEM((TAIL_LAST, D), jnp.float32),
            pltpu.SemaphoreType.DMA((NBUF,)),
            pltpu.SemaphoreType.DMA((NBUF,)),
            pltpu.SemaphoreType.DMA,
            pltpu.SemaphoreType.DMA,
            pltpu.SemaphoreType.DMA,
            pltpu.VMEM_SHARED((P, D), jnp.float32),
        ],
    )(feats, idx, table)


def kernel(unit_features, unit_position_ids, table):
    idx = unit_position_ids.astype(jnp.int32)
    return _run(unit_features, idx, table)
